# baseline (speedup 1.0000x reference)
import functools

import jax
import jax.numpy as jnp
from jax import lax
from jax.experimental import pallas as pl
from jax.experimental.pallas import tpu as pltpu
from jax.experimental.pallas import tpu_sc as plsc

_BN = 1000
_NSUB = 16
_NCORE = 2
_CHUNK = 128
_NPAD = 10240


def _mesh():
    return plsc.VectorSubcoreMesh(core_axis_name="c", subcore_axis_name="s",
                                  num_cores=_NCORE, num_subcores=_NSUB)


def _deg_sc(d3, zeros16, ones16, N):
    nsh = d3.shape[0]
    wrows = d3.shape[1]
    zr = _NPAD // _NSUB

    @functools.partial(
        pl.kernel,
        out_type=jax.ShapeDtypeStruct((_NCORE * _NPAD, 16), jnp.float32),
        mesh=_mesh(),
        scratch_types=[
            pltpu.VMEM_SHARED((_NPAD, 16), jnp.float32),
            pltpu.VMEM((wrows, _CHUNK), jnp.int32),
            pltpu.VMEM((_CHUNK, 16), jnp.float32),
        ],
    )
    def k(d_h, z_h, o_h, out_h, acc, d_buf, ones_b):
        core = lax.axis_index("c")
        sub = lax.axis_index("s")
        w = core * _NSUB + sub
        pltpu.sync_copy(d_h.at[w], d_buf)
        pltpu.sync_copy(o_h, ones_b)
        pltpu.sync_copy(z_h.at[pl.ds(sub * zr, zr)], acc.at[pl.ds(sub * zr, zr)])
        plsc.subcore_barrier()

        def body(t, carry):
            pltpu.sync_copy(ones_b, acc.at[d_buf.at[t]], add=True)
            return carry

        lax.fori_loop(0, wrows, body, 0)
        plsc.subcore_barrier()
        pltpu.sync_copy(acc.at[pl.ds(sub * zr, zr)],
                        out_h.at[pl.ds(core * _NPAD + sub * zr, zr)])

    return k(d3, zeros16, ones16)


def _spmm_sc(table, s3, d3, zeros, nc, N):
    srows = s3.shape[1]
    zr = _NPAD // _NSUB
    halves = nc // _NCORE

    @functools.partial(
        pl.kernel,
        out_type=[jax.ShapeDtypeStruct((_NPAD, _CHUNK), jnp.float32)
                  for _ in range(nc)],
        mesh=_mesh(),
        scratch_types=[
            pltpu.VMEM_SHARED((_NPAD, _CHUNK), jnp.float32),
            pltpu.VMEM((srows, _CHUNK), jnp.int32),
            pltpu.VMEM((srows, _CHUNK), jnp.int32),
            pltpu.VMEM((_CHUNK, _CHUNK), jnp.float32),
            pltpu.SemaphoreType.DMA,
        ],
    )
    def k(table_h, s_h, d_h, z_h, *rest):
        outs = rest[:nc]
        acc, s_buf, d_buf, rows, sem = rest[nc:]
        core = lax.axis_index("c")
        sub = lax.axis_index("s")
        pltpu.sync_copy(s_h.at[sub], s_buf)
        pltpu.sync_copy(d_h.at[sub], d_buf)

        def adjust(delta):
            def body(i, carry):
                j = i // 8
                kk = (i % 8) * 16
                s_buf[j, pl.ds(kk, 16)] = s_buf[j, pl.ds(kk, 16)] + delta
                return carry
            lax.fori_loop(0, srows * 8, body, 0)

        for half in range(halves):
            if half == 0:
                delta = core * N
            else:
                delta = jnp.int32(_NCORE * N)
            adjust(delta)
            pltpu.sync_copy(z_h.at[pl.ds(sub * zr, zr)],
                            acc.at[pl.ds(sub * zr, zr)])
            plsc.subcore_barrier()

            def edge_body(t, carry):
                pltpu.async_copy(table_h.at[s_buf.at[t]], rows, sem).wait()
                pltpu.sync_copy(rows, acc.at[d_buf.at[t]], add=True)
                return carry

            lax.fori_loop(0, srows, edge_body, 0)
            plsc.subcore_barrier()
            for cand in range(_NCORE):
                cc = half * _NCORE + cand

                @pl.when(core == cand)
                def _():
                    pltpu.sync_copy(acc.at[pl.ds(sub * zr, zr)],
                                    outs[cc].at[pl.ds(sub * zr, zr)])
            if half + 1 < halves:
                plsc.subcore_barrier()

    return k(table, s3, d3, zeros)


def _dinv_tc(degp, N):
    def body(dp_ref, o_ref):
        deg = dp_ref[:N, :] + dp_ref[_NPAD:_NPAD + N, :]
        dinv = 1.0 / jnp.sqrt(deg)
        o_ref[...] = jnp.broadcast_to(dinv[:, :1], (N, 128))

    return pl.pallas_call(
        body,
        out_shape=jax.ShapeDtypeStruct((N, 128), jnp.float32),
    )(degp)


def _prescale_tc(x, dinv, N, C):
    nc = C // 128
    G = N // _BN

    def body(x_ref, v_ref, o_ref):
        o_ref[...] = x_ref[...] * v_ref[:, :1]

    return pl.pallas_call(
        body,
        grid=(G, nc),
        in_specs=[
            pl.BlockSpec((_BN, 128), lambda i, c: (i, c)),
            pl.BlockSpec((_BN, 128), lambda i, c: (i, 0)),
        ],
        out_specs=pl.BlockSpec((_BN, 128), lambda i, c: (c * G + i, 0)),
        out_shape=jax.ShapeDtypeStruct((nc * N, 128), jnp.float32),
    )(x, dinv)


def _mm_stats_tc(aggs, dinv, W, b, N, Cin, Cout):
    nc = Cin // 128
    G = N // _BN

    def body(*refs):
        parts = refs[:nc]
        v_ref, w_ref, b_ref, h_ref, st_ref = refs[nc:]
        a = jnp.concatenate([p[...] for p in parts], axis=1) * v_ref[:, :1]
        h = jnp.dot(a, w_ref[...], preferred_element_type=jnp.float32,
                    precision=lax.Precision.HIGHEST) + b_ref[...]
        h_ref[...] = h
        cs = jnp.sum(h, axis=0, keepdims=True)
        cq = jnp.sum(h * h, axis=0, keepdims=True)
        st = jnp.concatenate([cs, cq, jnp.zeros((6, Cout), jnp.float32)], 0)
        st_ref[...] = st[None]

    in_specs = [pl.BlockSpec((_BN, 128), lambda i: (i, 0)) for _ in range(nc)]
    in_specs += [
        pl.BlockSpec((_BN, 128), lambda i: (i, 0)),
        pl.BlockSpec((Cin, Cout), lambda i: (0, 0)),
        pl.BlockSpec((1, Cout), lambda i: (0, 0)),
    ]
    return pl.pallas_call(
        body,
        grid=(G,),
        in_specs=in_specs,
        out_specs=[
            pl.BlockSpec((_BN, Cout), lambda i: (i, 0)),
            pl.BlockSpec((1, 8, Cout), lambda i: (i, 0, 0)),
        ],
        out_shape=[
            jax.ShapeDtypeStruct((N, Cout), jnp.float32),
            jax.ShapeDtypeStruct((G, 8, Cout), jnp.float32),
        ],
    )(*aggs, dinv, W, b)


def _graphnorm(h, st_ref, g_ref, be_ref, al_ref, N, eps=1e-5):
    m = jnp.sum(st_ref[:, 0, :], axis=0) * (1.0 / N)
    q = jnp.sum(st_ref[:, 1, :], axis=0) * (1.0 / N)
    al = al_ref[0]
    var = q - 2.0 * al * m * m + al * al * m * m
    xc = h - al * m
    return g_ref[0] * xc / jnp.sqrt(var + eps) + be_ref[0]


def _ep1_tc(hpre, st, g, be, al, dinv, N, C):
    nc = C // 128
    G = N // _BN

    def body(h_ref, st_ref, g_ref, be_ref, al_ref, v_ref, o_ref):
        y = _graphnorm(h_ref[...], st_ref, g_ref, be_ref, al_ref, N)
        o_ref[...] = jnp.maximum(y, 0.0) * v_ref[:, :1]

    return pl.pallas_call(
        body,
        grid=(G, nc),
        in_specs=[
            pl.BlockSpec((_BN, 128), lambda i, c: (i, c)),
            pl.BlockSpec((G, 8, 128), lambda i, c: (0, 0, c)),
            pl.BlockSpec((1, 128), lambda i, c: (0, c)),
            pl.BlockSpec((1, 128), lambda i, c: (0, c)),
            pl.BlockSpec((1, 128), lambda i, c: (0, c)),
            pl.BlockSpec((_BN, 128), lambda i, c: (i, 0)),
        ],
        out_specs=pl.BlockSpec((_BN, 128), lambda i, c: (c * G + i, 0)),
        out_shape=jax.ShapeDtypeStruct((nc * N, 128), jnp.float32),
    )(hpre, st, g, be, al, dinv)


def _ep2_tc(hpre, st, g, be, al, W3, dinv, N, C, Cout):
    G = N // _BN
    nco = Cout // 128

    def body(h_ref, st_ref, g_ref, be_ref, al_ref, w_ref, v_ref, o_ref):
        y = _graphnorm(h_ref[...], st_ref, g_ref, be_ref, al_ref, N)
        r = jnp.maximum(y, 0.0)
        t = jnp.dot(r, w_ref[...], preferred_element_type=jnp.float32,
                    precision=lax.Precision.HIGHEST)
        o_ref[...] = t * v_ref[:, :1]

    return pl.pallas_call(
        body,
        grid=(G, nco),
        in_specs=[
            pl.BlockSpec((_BN, C), lambda i, c: (i, 0)),
            pl.BlockSpec((G, 8, C), lambda i, c: (0, 0, 0)),
            pl.BlockSpec((1, C), lambda i, c: (0, 0)),
            pl.BlockSpec((1, C), lambda i, c: (0, 0)),
            pl.BlockSpec((1, C), lambda i, c: (0, 0)),
            pl.BlockSpec((C, 128), lambda i, c: (0, c)),
            pl.BlockSpec((_BN, 128), lambda i, c: (i, 0)),
        ],
        out_specs=pl.BlockSpec((_BN, 128), lambda i, c: (c * G + i, 0)),
        out_shape=jax.ShapeDtypeStruct((nco * N, 128), jnp.float32),
    )(hpre, st, g, be, al, W3, dinv)


def _final_tc(aggs, dinv, b3, N, C):
    nc = C // 128
    G = N // _BN

    def body(*refs):
        parts = refs[:nc]
        v_ref, b_ref, o_ref = refs[nc:]
        cols = [parts[c][...] * v_ref[:, :1] + b_ref[:, 128 * c:128 * (c + 1)]
                for c in range(nc)]
        o_ref[...] = jnp.concatenate(cols, axis=1)

    in_specs = [pl.BlockSpec((_BN, 128), lambda i: (i, 0)) for _ in range(nc)]
    in_specs += [
        pl.BlockSpec((_BN, 128), lambda i: (i, 0)),
        pl.BlockSpec((1, C), lambda i: (0, 0)),
    ]
    return pl.pallas_call(
        body,
        grid=(G,),
        in_specs=in_specs,
        out_specs=pl.BlockSpec((_BN, C), lambda i: (i, 0)),
        out_shape=jax.ShapeDtypeStruct((N, C), jnp.float32),
    )(*aggs, dinv, b3)


def kernel(x, edge_index, W1, b1, g1, be1, a1, W2, b2, g2, be2, a2, W3, b3):
    N, IN = x.shape
    H = W1.shape[1]
    OUT = W3.shape[1]
    E = edge_index.shape[1]

    nworker = _NCORE * _NSUB
    align = nworker * _CHUNK * 8
    EP = ((E + N + align - 1) // align) * align
    pad = EP - E - N

    loop = jnp.arange(N, dtype=jnp.int32)
    s_flat = jnp.concatenate(
        [edge_index[0], loop, jnp.zeros((pad,), jnp.int32)])
    d_flat = jnp.concatenate(
        [edge_index[1], loop, jnp.full((pad,), N, jnp.int32)])
    s3 = s_flat.reshape(_NSUB, -1, _CHUNK)
    d3 = d_flat.reshape(_NSUB, -1, _CHUNK)
    d3deg = d_flat.reshape(nworker, -1, _CHUNK)
    zeros = jnp.zeros((_NPAD, _CHUNK), jnp.float32)
    zeros16 = jnp.zeros((_NPAD, 16), jnp.float32)
    ones16 = jnp.ones((_CHUNK, 16), jnp.float32)

    degp = _deg_sc(d3deg, zeros16, ones16, N)
    dinv = _dinv_tc(degp, N)

    g0 = _prescale_tc(x, dinv, N, IN)
    a0 = _spmm_sc(g0, s3, d3, zeros, IN // 128, N)
    h1, st1 = _mm_stats_tc(a0, dinv, W1, b1.reshape(1, H), N, IN, H)
    g1c = _ep1_tc(h1, st1, g1.reshape(1, H), be1.reshape(1, H),
                  a1.reshape(1, H), dinv, N, H)
    a1g = _spmm_sc(g1c, s3, d3, zeros, H // 128, N)
    h2, st2 = _mm_stats_tc(a1g, dinv, W2, b2.reshape(1, H), N, H, H)
    g2c = _ep2_tc(h2, st2, g2.reshape(1, H), be2.reshape(1, H),
                  a2.reshape(1, H), W3, dinv, N, H, OUT)
    a2g = _spmm_sc(g2c, s3, d3, zeros, OUT // 128, N)
    return _final_tc(a2g, dinv, b3.reshape(1, OUT), N, OUT)
